# Initial kernel scaffold; baseline (speedup 1.0000x reference)
import jax
import jax.numpy as jnp
from jax.experimental import pallas as pl


def kernel(x_s, edge_index_s, edge_attr_s, batch_s, x_t, edge_index_t, edge_attr_t, batch_t, W_msg, b_msg, W_root):
    raise NotImplementedError("write your pallas kernel here")


# speedup vs baseline: 1.1821x; 1.1821x over previous
import functools

import jax
import jax.numpy as jnp
from jax import lax
from jax.experimental import pallas as pl
from jax.experimental.pallas import tpu as pltpu
from jax.experimental.pallas import tpu_sc as plsc

NC = 2
NS = 16
K = 80
KG = 8


def _mm_node_body(x_ref, w1_ref, wr_ref, y_ref, r_ref):
    x = x_ref[...]
    y_ref[...] = jnp.dot(x, w1_ref[...], preferred_element_type=jnp.float32)
    r_ref[...] = jnp.dot(x, wr_ref[...], preferred_element_type=jnp.float32)


def _mm_node(x, w1, wr, bn, half):
    n = x.shape[0]
    d = x.shape[1]
    nb = n // bn
    out = jax.ShapeDtypeStruct((NC * n, half), jnp.float32)
    return pl.pallas_call(
        _mm_node_body,
        grid=(nb, NC),
        in_specs=[
            pl.BlockSpec((bn, d), lambda i, c: (i, 0)),
            pl.BlockSpec((d, half), lambda i, c: (0, c)),
            pl.BlockSpec((d, half), lambda i, c: (0, c)),
        ],
        out_specs=[
            pl.BlockSpec((bn, half), lambda i, c, _nb=nb: (c * _nb + i, 0)),
            pl.BlockSpec((bn, half), lambda i, c, _nb=nb: (c * _nb + i, 0)),
        ],
        out_shape=[out, out],
    )(x, w1, wr)


def _mm_edge_body(ea_ref, w2_ref, b_ref, e_ref):
    e_ref[...] = (
        jnp.dot(ea_ref[...], w2_ref[...], preferred_element_type=jnp.float32)
        + b_ref[...]
    )


def _mm_edge(ea, w2, b, be, half):
    e = ea.shape[0]
    de = ea.shape[1]
    eb = e // be
    return pl.pallas_call(
        _mm_edge_body,
        grid=(eb, NC),
        in_specs=[
            pl.BlockSpec((be, de), lambda i, c: (i, 0)),
            pl.BlockSpec((de, half), lambda i, c: (0, c)),
            pl.BlockSpec((1, half), lambda i, c: (0, c)),
        ],
        out_specs=pl.BlockSpec((be, half), lambda i, c, _eb=eb: (c * _eb + i, 0)),
        out_shape=jax.ShapeDtypeStruct((NC * e, half), jnp.float32),
    )(ea, w2, b.reshape(1, -1))


def _merge_body(a_ref, h_ref):
    h_ref[...] = jnp.maximum(a_ref[...], 0.0)


def _merge_relu(agg2, n, bn, half):
    nb = n // bn
    return pl.pallas_call(
        _merge_body,
        grid=(nb, NC),
        in_specs=[pl.BlockSpec((bn, half), lambda i, c, _nb=nb: (c * _nb + i, 0))],
        out_specs=pl.BlockSpec((bn, half), lambda i, c: (i, c)),
        out_shape=jax.ShapeDtypeStruct((n, NC * half), jnp.float32),
    )(agg2)


def _sc_edge_kernel(n, e, half, kchunks):
    eps = e // NS
    rows = (n // NS) // 8 * 8
    rem = n - NS * rows
    mesh = plsc.VectorSubcoreMesh(core_axis_name="c", subcore_axis_name="s")

    @functools.partial(
        pl.kernel,
        out_type=jax.ShapeDtypeStruct((NC * n, half), jnp.float32),
        mesh=mesh,
        scratch_types=[
            pltpu.VMEM((KG, K), jnp.int32),
            pltpu.VMEM((KG, K), jnp.int32),
            pltpu.VMEM((K, half), jnp.float32),
            pltpu.VMEM((K, half), jnp.float32),
            pltpu.VMEM_SHARED((n, half), jnp.float32),
            pltpu.SemaphoreType.DMA,
        ],
    )
    def body(src_hbm, dst_hbm, y_hbm, e_hbm, r_hbm, out_hbm,
             src_v, dst_v, g_v, e_v, acc, sem):
        c = lax.axis_index("c")
        s = lax.axis_index("s")
        pltpu.sync_copy(r_hbm.at[pl.ds(c * n + s * rows, rows)],
                        acc.at[pl.ds(s * rows, rows)])
        if rem:
            @pl.when(s == 0)
            def _():
                pltpu.sync_copy(r_hbm.at[pl.ds(c * n + NS * rows, rem)],
                                acc.at[pl.ds(NS * rows, rem)])
        plsc.subcore_barrier()

        def chunk(k, carry):
            g = k // KG
            m = k % KG

            @pl.when(m == 0)
            def _():
                pltpu.sync_copy(src_hbm.at[c].at[s].at[pl.ds(g * KG, KG)],
                                src_v)
                pltpu.sync_copy(dst_hbm.at[s].at[pl.ds(g * KG, KG)], dst_v)

            base = c * e + s * eps + k * K
            pltpu.sync_copy(e_hbm.at[pl.ds(base, K)], e_v)
            pltpu.async_copy(y_hbm.at[src_v.at[m]], g_v, sem).wait()

            def row(i, carry2):
                for j in range(half // 16):
                    sl = pl.ds(j * 16, 16)
                    g_v[i, sl] = jnp.maximum(g_v[i, sl] + e_v[i, sl], 0.0)
                return carry2

            lax.fori_loop(0, K, row, 0, unroll=2)
            pltpu.sync_copy(g_v, acc.at[dst_v.at[m]], add=True)
            return carry

        lax.fori_loop(0, kchunks, chunk, 0)
        plsc.subcore_barrier()
        pltpu.sync_copy(acc.at[pl.ds(s * rows, rows)],
                        out_hbm.at[pl.ds(c * n + s * rows, rows)])
        if rem:
            @pl.when(s == 0)
            def _():
                pltpu.sync_copy(acc.at[pl.ds(NS * rows, rem)],
                                out_hbm.at[pl.ds(c * n + NS * rows, rem)])

    return body


def _psi1(x, edge_index, edge_attr, w1, w2, b, wr):
    n, d = x.shape
    e = edge_index.shape[1]
    half = d // NC
    eps = e // NS
    kchunks = eps // K

    src = edge_index[0]
    dst = edge_index[1]
    kc_pad = (kchunks + KG - 1) // KG * KG
    src_pc = (src.reshape(1, NS, kchunks, K)
              + jnp.arange(NC, dtype=jnp.int32).reshape(NC, 1, 1, 1) * n)
    src_pc = jnp.pad(src_pc, ((0, 0), (0, 0), (0, kc_pad - kchunks), (0, 0)))
    dst3 = jnp.pad(dst.reshape(1, NS, kchunks, K),
                   ((0, 0), (0, 0), (0, kc_pad - kchunks), (0, 0)))[0]

    y2, r2 = _mm_node(x, w1, wr, 1000, half)
    e2 = _mm_edge(edge_attr, w2, b, 2000, half)
    agg2 = _sc_edge_kernel(n, e, half, kchunks)(src_pc, dst3, y2, e2, r2)
    return _merge_relu(agg2, n, 1000, half)


def kernel(x_s, edge_index_s, edge_attr_s, batch_s,
           x_t, edge_index_t, edge_attr_t, batch_t,
           W_msg, b_msg, W_root):
    d = x_s.shape[1]
    w1 = W_msg[:d]
    w2 = W_msg[d:]
    h_s = _psi1(x_s, edge_index_s, edge_attr_s, w1, w2, b_msg, W_root)
    h_t = _psi1(x_t, edge_index_t, edge_attr_t, w1, w2, b_msg, W_root)
    return (h_s, h_t)


# speedup vs baseline: 1.7316x; 1.4649x over previous
import functools

import jax
import jax.numpy as jnp
from jax import lax
from jax.experimental import pallas as pl
from jax.experimental.pallas import tpu as pltpu
from jax.experimental.pallas import tpu_sc as plsc

NC = 2
NS = 16
K = 40
KG = 8
NB = 3


def _mm_node_body(x_ref, w1_ref, wr_ref, y_ref, r_ref):
    x = x_ref[...]
    y_ref[...] = jnp.dot(x, w1_ref[...], preferred_element_type=jnp.float32)
    r_ref[...] = jnp.dot(x, wr_ref[...], preferred_element_type=jnp.float32)


def _mm_node(x, w1, wr, bn, half):
    n = x.shape[0]
    d = x.shape[1]
    nb = n // bn
    out = jax.ShapeDtypeStruct((NC * n, half), jnp.float32)
    return pl.pallas_call(
        _mm_node_body,
        grid=(nb, NC),
        in_specs=[
            pl.BlockSpec((bn, d), lambda i, c: (i, 0)),
            pl.BlockSpec((d, half), lambda i, c: (0, c)),
            pl.BlockSpec((d, half), lambda i, c: (0, c)),
        ],
        out_specs=[
            pl.BlockSpec((bn, half), lambda i, c, _nb=nb: (c * _nb + i, 0)),
            pl.BlockSpec((bn, half), lambda i, c, _nb=nb: (c * _nb + i, 0)),
        ],
        out_shape=[out, out],
    )(x, w1, wr)


def _mm_edge_body(ea_ref, w2_ref, b_ref, e_ref):
    e_ref[...] = (
        jnp.dot(ea_ref[...], w2_ref[...], preferred_element_type=jnp.float32)
        + b_ref[...]
    )


def _mm_edge(ea, w2, b, be, half):
    e = ea.shape[0]
    de = ea.shape[1]
    eb = e // be
    return pl.pallas_call(
        _mm_edge_body,
        grid=(eb, NC),
        in_specs=[
            pl.BlockSpec((be, de), lambda i, c: (i, 0)),
            pl.BlockSpec((de, half), lambda i, c: (0, c)),
            pl.BlockSpec((1, half), lambda i, c: (0, c)),
        ],
        out_specs=pl.BlockSpec((be, half), lambda i, c, _eb=eb: (c * _eb + i, 0)),
        out_shape=jax.ShapeDtypeStruct((NC * e, half), jnp.float32),
    )(ea, w2, b.reshape(1, -1))


def _merge_body(a_ref, h_ref):
    h_ref[...] = jnp.maximum(a_ref[...], 0.0)


def _merge_relu(agg2, n, bn, half):
    nb = n // bn
    return pl.pallas_call(
        _merge_body,
        grid=(nb, NC),
        in_specs=[pl.BlockSpec((bn, half), lambda i, c, _nb=nb: (c * _nb + i, 0))],
        out_specs=pl.BlockSpec((bn, half), lambda i, c: (i, c)),
        out_shape=jax.ShapeDtypeStruct((n, NC * half), jnp.float32),
    )(agg2)


def _sc_edge_kernel(n, e, half, kchunks):
    eps = e // NS
    rows = (n // NS) // 8 * 8
    rem = n - NS * rows
    mesh = plsc.VectorSubcoreMesh(core_axis_name="c", subcore_axis_name="s")

    @functools.partial(
        pl.kernel,
        out_type=jax.ShapeDtypeStruct((NC * n, half), jnp.float32),
        mesh=mesh,
        scratch_types=[
            pltpu.VMEM((2, KG, K), jnp.int32),
            pltpu.VMEM((2, KG, K), jnp.int32),
            *[pltpu.VMEM((K, half), jnp.float32) for _ in range(NB)],
            *[pltpu.VMEM((K, half), jnp.float32) for _ in range(NB)],
            pltpu.VMEM_SHARED((n, half), jnp.float32),
            *[pltpu.SemaphoreType.DMA for _ in range(3 * NB)],
        ],
    )
    def body(src_hbm, dst_hbm, y_hbm, e_hbm, r_hbm, out_hbm,
             src_v, dst_v, *rest):
        gb = rest[:NB]
        eb = rest[NB:2 * NB]
        acc = rest[2 * NB]
        sg = rest[2 * NB + 1:2 * NB + 1 + NB]
        se = rest[2 * NB + 1 + NB:2 * NB + 1 + 2 * NB]
        ss = rest[2 * NB + 1 + 2 * NB:]
        c = lax.axis_index("c")
        s = lax.axis_index("s")
        pltpu.sync_copy(r_hbm.at[pl.ds(c * n + s * rows, rows)],
                        acc.at[pl.ds(s * rows, rows)])
        if rem:
            @pl.when(s == 0)
            def _():
                pltpu.sync_copy(r_hbm.at[pl.ds(c * n + NS * rows, rem)],
                                acc.at[pl.ds(NS * rows, rem)])
        plsc.subcore_barrier()

        def issue(k, b):
            grp = k // KG
            gp = grp % 2
            m = k % KG

            @pl.when(m == 0)
            def _():
                pltpu.sync_copy(src_hbm.at[c].at[s].at[pl.ds(grp * KG, KG)],
                                src_v.at[gp])
                pltpu.sync_copy(dst_hbm.at[s].at[pl.ds(grp * KG, KG)],
                                dst_v.at[gp])

            base = c * e + s * eps + k * K
            pltpu.async_copy(e_hbm.at[pl.ds(base, K)], eb[b], se[b])
            pltpu.async_copy(y_hbm.at[src_v.at[gp].at[m]], gb[b], sg[b])

        def drain(ref, sem):
            pltpu.make_async_copy(y_hbm.at[pl.ds(0, K)], ref, sem).wait()

        def compute(b):
            g_v, e_v = gb[b], eb[b]

            def row(i, carry2):
                for j in range(half // 16):
                    sl = pl.ds(j * 16, 16)
                    g_v[i, sl] = jnp.maximum(g_v[i, sl] + e_v[i, sl], 0.0)
                return carry2

            lax.fori_loop(0, K, row, 0, unroll=4)

        def scatter(k, b):
            grp = k // KG
            gp = grp % 2
            m = k % KG
            pltpu.async_copy(gb[b], acc.at[dst_v.at[gp].at[m]], ss[b],
                             add=True)

        assert kchunks % NB == 1
        triples = kchunks // NB
        issue(0, 0)
        issue(1, 1)

        def triple(t, carry):
            for j in range(NB):
                k = t * NB + j
                nb = (j + 2) % NB
                drain(gb[j], sg[j])
                drain(eb[j], se[j])
                compute(j)
                if j == 0:
                    @pl.when(t > 0)
                    def _():
                        drain(gb[nb], ss[nb])
                        issue(k + 2, nb)

                    @pl.when(t == 0)
                    def _():
                        issue(k + 2, nb)
                else:
                    drain(gb[nb], ss[nb])

                    @pl.when(k + 2 < kchunks)
                    def _():
                        issue(k + 2, nb)
                scatter(k, j)
            return carry

        lax.fori_loop(0, triples, triple, 0)
        klast = kchunks - 1
        blast = klast % NB
        drain(gb[blast], sg[blast])
        drain(eb[blast], se[blast])
        compute(blast)
        scatter(klast, blast)
        for j in range(NB):
            if j != (blast + 1) % NB:
                drain(gb[j], ss[j])
        plsc.subcore_barrier()
        pltpu.sync_copy(acc.at[pl.ds(s * rows, rows)],
                        out_hbm.at[pl.ds(c * n + s * rows, rows)])
        if rem:
            @pl.when(s == 0)
            def _():
                pltpu.sync_copy(acc.at[pl.ds(NS * rows, rem)],
                                out_hbm.at[pl.ds(c * n + NS * rows, rem)])

    return body


def _psi1(x, edge_index, edge_attr, w1, w2, b, wr):
    n, d = x.shape
    e = edge_index.shape[1]
    half = d // NC
    eps = e // NS
    kchunks = eps // K

    src = edge_index[0]
    dst = edge_index[1]
    kc_pad = (kchunks + KG - 1) // KG * KG
    src_pc = (src.reshape(1, NS, kchunks, K)
              + jnp.arange(NC, dtype=jnp.int32).reshape(NC, 1, 1, 1) * n)
    src_pc = jnp.pad(src_pc, ((0, 0), (0, 0), (0, kc_pad - kchunks), (0, 0)))
    dst3 = jnp.pad(dst.reshape(1, NS, kchunks, K),
                   ((0, 0), (0, 0), (0, kc_pad - kchunks), (0, 0)))[0]

    y2, r2 = _mm_node(x, w1, wr, 1000, half)
    e2 = _mm_edge(edge_attr, w2, b, 2000, half)
    agg2 = _sc_edge_kernel(n, e, half, kchunks)(src_pc, dst3, y2, e2, r2)
    return _merge_relu(agg2, n, 1000, half)


def kernel(x_s, edge_index_s, edge_attr_s, batch_s,
           x_t, edge_index_t, edge_attr_t, batch_t,
           W_msg, b_msg, W_root):
    d = x_s.shape[1]
    w1 = W_msg[:d]
    w2 = W_msg[d:]
    h_s = _psi1(x_s, edge_index_s, edge_attr_s, w1, w2, b_msg, W_root)
    h_t = _psi1(x_t, edge_index_t, edge_attr_t, w1, w2, b_msg, W_root)
    return (h_s, h_t)
